# baseline (speedup 1.0000x reference)
import jax
import jax.numpy as jnp
from jax.experimental import pallas as pl

_B = 128
_N = 8192
_BR = 64


def _onehot_body(x_ref, o_ref):
    x = x_ref[...]
    m = jnp.max(x, axis=1, keepdims=True)
    iota = jax.lax.broadcasted_iota(jnp.int32, x.shape, 1)
    cand = jnp.where(x == m, iota, _N)
    idx = jnp.min(cand, axis=1, keepdims=True)
    o_ref[...] = (iota == idx).astype(jnp.float32)


def kernel(coords):
    return pl.pallas_call(
        _onehot_body,
        out_shape=jax.ShapeDtypeStruct((_B, _N), jnp.float32),
        grid=(_B // _BR,),
        in_specs=[pl.BlockSpec((_BR, _N), lambda i: (i, 0))],
        out_specs=pl.BlockSpec((_BR, _N), lambda i: (i, 0)),
    )(coords)


# speedup vs baseline: 1.0256x; 1.0256x over previous
import jax
import jax.numpy as jnp
from jax import lax
from jax.experimental import pallas as pl
from jax.experimental.pallas import tpu as pltpu

_B = 128
_N = 8192
_NCH = 4
_CH = _B // _NCH


def _body(x_hbm, o_hbm, xv, ov, rsem, wsem):
    iota = lax.broadcasted_iota(jnp.int32, (_CH, _N), 1)
    in_cp = []
    for c in range(_NCH):
        cp = pltpu.make_async_copy(
            x_hbm.at[pl.ds(c * _CH, _CH)], xv.at[pl.ds(c * _CH, _CH)],
            rsem.at[c],
        )
        cp.start()
        in_cp.append(cp)
    out_cp = []
    for c in range(_NCH):
        in_cp[c].wait()
        x = xv[pl.ds(c * _CH, _CH), :]
        m = jnp.max(x, axis=1, keepdims=True)
        cand = jnp.where(x == m, iota, _N)
        idx = jnp.min(cand, axis=1, keepdims=True)
        ov[pl.ds(c * _CH, _CH), :] = (iota == idx).astype(jnp.float32)
        cp = pltpu.make_async_copy(
            ov.at[pl.ds(c * _CH, _CH)], o_hbm.at[pl.ds(c * _CH, _CH)],
            wsem.at[c],
        )
        cp.start()
        out_cp.append(cp)
    for cp in out_cp:
        cp.wait()


def kernel(coords):
    return pl.pallas_call(
        _body,
        out_shape=jax.ShapeDtypeStruct((_B, _N), jnp.float32),
        in_specs=[pl.BlockSpec(memory_space=pl.ANY)],
        out_specs=pl.BlockSpec(memory_space=pl.ANY),
        scratch_shapes=[
            pltpu.VMEM((_B, _N), jnp.float32),
            pltpu.VMEM((_B, _N), jnp.float32),
            pltpu.SemaphoreType.DMA((_NCH,)),
            pltpu.SemaphoreType.DMA((_NCH,)),
        ],
    )(coords)
